# baseline (speedup 1.0000x reference)
import jax
import jax.numpy as jnp
from jax.experimental import pallas as pl
from jax.experimental.pallas import tpu as pltpu

_GH = 32
_GW = 32
_NT = _GH * _GW
_N = _NT + 1
_D = 2 * _GW - 1
_F = _D * _D
_BP = (_D - 1) * _D + _GW


def _rpe_body(att_ref, w_ref, cls_ref, out_ref, bp_ref, kf_ref):
    for xi in range(_GW):
        s = _GW - 1 - xi
        bp_ref[xi:xi + 1, :] = w_ref[0, 0:1, s:s + _BP]
    for r in range(_D):
        kf_ref[:, r * _GW:(r + 1) * _GW] = bp_ref[:, r * _D:r * _D + _GW]

    cls2tok = cls_ref[0, 0, 0]
    tok2cls = cls_ref[0, 0, 1]
    cls2cls = cls_ref[0, 0, 2]
    for b in range(att_ref.shape[0]):
        out_ref[b, 0, :, 0:1] = att_ref[b, 0, :, 0:1] + tok2cls
        out_ref[b, 0, 0:1, 1:_N] = att_ref[b, 0, 0:1, 1:_N] + cls2tok
        out_ref[b, 0, 0:1, 0:1] = att_ref[b, 0, 0:1, 0:1] + cls2cls
        for yi in range(_GH):
            r0 = 1 + _GW * yi
            off = (_GH - 1 - yi) * _GW
            bias = kf_ref[:, off:off + _NT]
            out_ref[b, 0, r0:r0 + _GW, 1:_N] = (
                att_ref[b, 0, r0:r0 + _GW, 1:_N] + bias)


def kernel(attention_tensor, ref_bias_lut, patch_grid_hw):
    grid_hw = jnp.asarray(patch_grid_hw).astype(jnp.int32)
    d = (grid_hw[0] - _GH) + (grid_hw[1] - _GW)
    num_heads = ref_bias_lut.shape[1]
    lut_t = ref_bias_lut.T
    tok_t = jax.lax.dynamic_slice_in_dim(lut_t, d, _F, axis=1)
    cls_t = jax.lax.dynamic_slice_in_dim(lut_t, _F + d, 3, axis=1)
    revb = tok_t.reshape(num_heads, _D, _D)[:, ::-1, ::-1]
    w = revb.reshape(num_heads, 1, _F)
    cls_t = cls_t.reshape(num_heads, 1, 3)
    batch = attention_tensor.shape[0]
    return pl.pallas_call(
        _rpe_body,
        grid=(num_heads,),
        in_specs=[
            pl.BlockSpec((batch, 1, _N, _N), lambda h: (0, h, 0, 0)),
            pl.BlockSpec((1, 1, _F), lambda h: (h, 0, 0)),
            pl.BlockSpec((1, 1, 3), lambda h: (h, 0, 0)),
        ],
        out_specs=pl.BlockSpec((batch, 1, _N, _N), lambda h: (0, h, 0, 0)),
        out_shape=jax.ShapeDtypeStruct(attention_tensor.shape,
                                       attention_tensor.dtype),
        scratch_shapes=[
            pltpu.VMEM((_GW, _BP), jnp.float32),
            pltpu.VMEM((_GW, _D * _GW), jnp.float32),
        ],
        compiler_params=pltpu.CompilerParams(
            dimension_semantics=("parallel",)),
    )(attention_tensor, w, cls_t)
